# baseline (device time: 62413 ns/iter reference)
import jax
import jax.numpy as jnp
from jax import lax
from jax.experimental import pallas as pl
from jax.experimental.pallas import tpu as pltpu

K = 16


def kernel(x):
    m_per, n = x.shape
    half = m_per // 2
    chunk = half // K

    def body(x_ref, out_ref, xstage, mine_ref, recv_ref,
             y_send, y_recv, z_send, z_recv, load_sem, store_sem):
        my_x = lax.axis_index("x")
        my_y = lax.axis_index("y")
        my_z = lax.axis_index("z")
        y_nbr = (my_x, 1 - my_y, my_z)
        z_nbr = (my_x, my_y, 1 - my_z)

        loads = []
        for i in range(K):
            row = my_z * half + i * chunk
            ld = pltpu.make_async_copy(
                x_ref.at[pl.ds(row, chunk), :],
                xstage.at[pl.ds(row, chunk), :],
                load_sem.at[i],
            )
            ld.start()
            loads.append(ld)
        row2 = (1 - my_z) * half
        ld_other = pltpu.make_async_copy(
            x_ref.at[pl.ds(row2, half), :],
            xstage.at[pl.ds(row2, half), :],
            load_sem.at[K],
        )
        ld_other.start()

        barrier_sem = pltpu.get_barrier_semaphore()
        for nbr in (y_nbr, z_nbr):
            pl.semaphore_signal(
                barrier_sem, inc=1, device_id=nbr,
                device_id_type=pl.DeviceIdType.MESH,
            )
        pl.semaphore_wait(barrier_sem, 2)

        own = my_y * m_per
        other = (1 - my_y) * m_per

        y_sends = []
        for i in range(K):
            row = my_z * half + i * chunk
            loads[i].wait()
            mine_ref[pl.ds(row, chunk), :] = (
                xstage[pl.ds(row, chunk), :].astype(jnp.bfloat16)
            )
            r = pltpu.make_async_remote_copy(
                src_ref=mine_ref.at[pl.ds(row, chunk), :],
                dst_ref=recv_ref.at[pl.ds(row, chunk), :],
                send_sem=y_send.at[i],
                recv_sem=y_recv.at[i],
                device_id=y_nbr,
                device_id_type=pl.DeviceIdType.MESH,
            )
            r.start()
            y_sends.append(r)

        ld_other.wait()
        mine_ref[pl.ds(row2, half), :] = (
            xstage[pl.ds(row2, half), :].astype(jnp.bfloat16)
        )
        mine_store = pltpu.make_async_copy(
            mine_ref, out_ref.at[pl.ds(own, m_per), :], store_sem.at[0]
        )
        mine_store.start()

        z_sends = []
        for i in range(K):
            row = my_z * half + i * chunk
            yr = pltpu.make_async_remote_copy(
                src_ref=recv_ref.at[pl.ds(row, chunk), :],
                dst_ref=recv_ref.at[pl.ds(row, chunk), :],
                send_sem=y_send.at[i],
                recv_sem=y_recv.at[i],
                device_id=y_nbr,
                device_id_type=pl.DeviceIdType.MESH,
            )
            yr.wait_recv()
            zr = pltpu.make_async_remote_copy(
                src_ref=recv_ref.at[pl.ds(row, chunk), :],
                dst_ref=recv_ref.at[pl.ds(row, chunk), :],
                send_sem=z_send.at[i],
                recv_sem=z_recv.at[i],
                device_id=z_nbr,
                device_id_type=pl.DeviceIdType.MESH,
            )
            zr.start()
            z_sends.append(zr)

        y_store = pltpu.make_async_copy(
            recv_ref.at[pl.ds(my_z * half, half), :],
            out_ref.at[pl.ds(other + my_z * half, half), :],
            store_sem.at[1],
        )
        y_store.start()

        for i in range(K):
            row = (1 - my_z) * half + i * chunk
            zrec = pltpu.make_async_remote_copy(
                src_ref=recv_ref.at[pl.ds(row, chunk), :],
                dst_ref=recv_ref.at[pl.ds(row, chunk), :],
                send_sem=z_send.at[i],
                recv_sem=z_recv.at[i],
                device_id=z_nbr,
                device_id_type=pl.DeviceIdType.MESH,
            )
            zrec.wait_recv()

        z_store = pltpu.make_async_copy(
            recv_ref.at[pl.ds((1 - my_z) * half, half), :],
            out_ref.at[pl.ds(other + (1 - my_z) * half, half), :],
            store_sem.at[2],
        )
        z_store.start()

        for i in range(K):
            y_sends[i].wait_send()
            z_sends[i].wait_send()
        mine_store.wait()
        y_store.wait()
        z_store.wait()

    return pl.pallas_call(
        body,
        out_shape=jax.ShapeDtypeStruct((2 * m_per, n), jnp.bfloat16),
        in_specs=[pl.BlockSpec(memory_space=pltpu.MemorySpace.HBM)],
        out_specs=pl.BlockSpec(memory_space=pltpu.MemorySpace.HBM),
        scratch_shapes=[
            pltpu.VMEM((m_per, n), jnp.float32),
            pltpu.VMEM((m_per, n), jnp.bfloat16),
            pltpu.VMEM((m_per, n), jnp.bfloat16),
            pltpu.SemaphoreType.DMA((K,)),
            pltpu.SemaphoreType.DMA((K,)),
            pltpu.SemaphoreType.DMA((K,)),
            pltpu.SemaphoreType.DMA((K,)),
            pltpu.SemaphoreType.DMA((K + 1,)),
            pltpu.SemaphoreType.DMA((3,)),
        ],
        compiler_params=pltpu.CompilerParams(collective_id=0),
    )(x)
